# baseline (speedup 1.0000x reference)
import functools

import jax
import jax.numpy as jnp
from jax import lax
from jax.experimental import pallas as pl
from jax.experimental.pallas import tpu as pltpu
from jax.experimental.pallas import tpu_sc as plsc

B = 4096
L = 300
DIM = 64

_info = plsc.get_sparse_core_info()
NC = _info.num_cores
NS = _info.num_subcores
NW = NC * NS
BW = B // NW


def _gather_body(table_hbm, idxt_hbm, out_hbm, idx_v, rows0, rows1, sem0, sem1):
    wid = lax.axis_index("s") * NC + lax.axis_index("c")
    b0 = wid * BW

    pltpu.sync_copy(idxt_hbm.at[:, pl.ds(b0, BW)], idx_v)

    bufs = (rows0, rows1)
    sems = (sem0, sem1)

    def fire(l, p):
        pltpu.async_copy(table_hbm.at[idx_v.at[l]], bufs[p], sems[p])

    def drain_store(l, p):
        pltpu.make_async_copy(
            out_hbm.at[pl.ds(b0, BW), l], bufs[p], sems[p]
        ).wait()
        pltpu.sync_copy(bufs[p], out_hbm.at[pl.ds(b0, BW), l])

    fire(0, 0)

    def pair(k, carry):
        l_odd = 2 * k + 1
        fire(l_odd, 1)
        drain_store(l_odd - 1, 0)
        l_even = 2 * k + 2
        fire(l_even, 0)
        drain_store(l_even - 1, 1)
        return carry

    lax.fori_loop(0, (L - 2) // 2, pair, 0)
    fire(L - 1, 1)
    drain_store(L - 2, 0)
    drain_store(L - 1, 1)


@functools.partial(
    pl.kernel,
    mesh=plsc.VectorSubcoreMesh(core_axis_name="c", subcore_axis_name="s"),
    out_type=jax.ShapeDtypeStruct((B, L, DIM), jnp.float32),
    scratch_types=[
        pltpu.VMEM((L, BW), jnp.int32),
        pltpu.VMEM((BW, DIM), jnp.float32),
        pltpu.VMEM((BW, DIM), jnp.float32),
        pltpu.SemaphoreType.DMA,
        pltpu.SemaphoreType.DMA,
    ],
    compiler_params=pltpu.CompilerParams(use_tc_tiling_on_sc=False),
)
def _embed_gather(table_hbm, idxt_hbm, out_hbm, idx_v, rows0, rows1, sem0, sem1):
    _gather_body(table_hbm, idxt_hbm, out_hbm, idx_v, rows0, rows1, sem0, sem1)


def kernel(input_ids, attention_mask, table):
    embeds = _embed_gather(table, input_ids.T)
    return (embeds, embeds, attention_mask)
